# baseline (speedup 1.0000x reference)
import jax
import jax.numpy as jnp
from jax import lax
from jax.experimental import pallas as pl
from jax.experimental.pallas import tpu as pltpu

B = 16384
D = 16
R = 5
TILE = 16384


def _tc_body(gu_ref, gi_ref, pcat_ref, w_ref, relrow_ref, xui_ref, pui_ref):
    gu = gu_ref[...]
    gi = gi_ref[...]
    t = jnp.dot(gu, pcat_ref[...], preferred_element_type=jnp.float32)
    e = t * jnp.concatenate([gi, gi], axis=1)
    pstk = jnp.dot(e, w_ref[...], preferred_element_type=jnp.float32)
    valid = lax.broadcasted_iota(jnp.int32, (1, 8), 1) < R
    neg_inf = jnp.float32(float("-inf"))
    mx = jnp.max(jnp.where(valid, pstk, neg_inf), axis=1, keepdims=True)
    ex = jnp.where(valid, jnp.exp(pstk - mx), 0.0)
    den = jnp.sum(ex, axis=1, keepdims=True)
    num = jnp.sum(relrow_ref[...] * ex, axis=1, keepdims=True)
    xui_ref[...] = (num / den)[:, 0]
    pui_ref[...] = pstk[:, :R]


@jax.jit
def _tc_call(gu, gi, pcat, w, relrow):
    grid = (B // TILE,)
    return pl.pallas_call(
        _tc_body,
        grid=grid,
        in_specs=[
            pl.BlockSpec((TILE, D), lambda b: (b, 0)),
            pl.BlockSpec((TILE, D), lambda b: (b, 0)),
            pl.BlockSpec((D, 2 * D), lambda b: (0, 0)),
            pl.BlockSpec((2 * D, 8), lambda b: (0, 0)),
            pl.BlockSpec((1, 8), lambda b: (0, 0)),
        ],
        out_specs=[
            pl.BlockSpec((TILE,), lambda b: (b,)),
            pl.BlockSpec((TILE, R), lambda b: (b, 0)),
        ],
        out_shape=[
            jax.ShapeDtypeStruct((B,), jnp.float32),
            jax.ShapeDtypeStruct((B, R), jnp.float32),
        ],
        compiler_params=pltpu.CompilerParams(
            dimension_semantics=("arbitrary",)),
    )(gu, gi, pcat, w, relrow)


def kernel(gu, gi, P, A, relations):
    gu = jnp.squeeze(gu)
    gi = jnp.squeeze(gi)
    pcat = jnp.concatenate([P[0], P[1]], axis=1)
    w = jnp.zeros((2 * D, 8), jnp.float32)
    w = w.at[:D, :R].set(jnp.broadcast_to(A[:, 0][None, :], (D, R)))
    w = w.at[D:, :R].set(jnp.broadcast_to(A[:, 1][None, :], (D, R)))
    relrow = jnp.zeros((1, 8), jnp.float32).at[0, :R].set(relations)
    return _tc_call(gu, gi, pcat, w, relrow)


# speedup vs baseline: 4.2768x; 4.2768x over previous
import jax
import jax.numpy as jnp
from jax import lax
from jax.experimental import pallas as pl
from jax.experimental.pallas import tpu as pltpu

B = 16384
D = 16
R = 5
TILE = 2048


def _tc_body(gut_ref, git_ref, pt_ref, apad_ref, relcol_ref, xui_ref, pui_ref):
    gut = gut_ref[...]
    git = git_ref[...]
    t0 = jnp.dot(pt_ref[0], gut, preferred_element_type=jnp.float32)
    t1 = jnp.dot(pt_ref[1], gut, preferred_element_type=jnp.float32)
    m0 = jnp.sum(t0 * git, axis=0, keepdims=True)
    m1 = jnp.sum(t1 * git, axis=0, keepdims=True)
    mstk = jnp.concatenate([m0, m1], axis=0)
    pstk = jnp.dot(apad_ref[...], mstk, preferred_element_type=jnp.float32)
    valid = lax.broadcasted_iota(jnp.int32, (8, 1), 0) < R
    neg_inf = jnp.float32(float("-inf"))
    mx = jnp.max(jnp.where(valid, pstk, neg_inf), axis=0, keepdims=True)
    ex = jnp.where(valid, jnp.exp(pstk - mx), 0.0)
    den = jnp.sum(ex, axis=0, keepdims=True)
    num = jnp.sum(relcol_ref[...] * ex, axis=0, keepdims=True)
    xui_ref[...] = num / den
    pui_ref[...] = pstk[:R]


@jax.jit
def _tc_call(gut, git, pt, apad, relcol):
    grid = (B // TILE,)
    return pl.pallas_call(
        _tc_body,
        grid=grid,
        in_specs=[
            pl.BlockSpec((D, TILE), lambda b: (0, b)),
            pl.BlockSpec((D, TILE), lambda b: (0, b)),
            pl.BlockSpec((2, D, D), lambda b: (0, 0, 0)),
            pl.BlockSpec((8, 2), lambda b: (0, 0)),
            pl.BlockSpec((8, 1), lambda b: (0, 0)),
        ],
        out_specs=[
            pl.BlockSpec((1, TILE), lambda b: (0, b)),
            pl.BlockSpec((R, TILE), lambda b: (0, b)),
        ],
        out_shape=[
            jax.ShapeDtypeStruct((1, B), jnp.float32),
            jax.ShapeDtypeStruct((R, B), jnp.float32),
        ],
        compiler_params=pltpu.CompilerParams(
            dimension_semantics=("arbitrary",)),
    )(gut, git, pt, apad, relcol)


def kernel(gu, gi, P, A, relations):
    gu = jnp.squeeze(gu)
    gi = jnp.squeeze(gi)
    gut = gu.T
    git = gi.T
    pt = jnp.swapaxes(P, 1, 2)
    apad = jnp.zeros((8, 2), jnp.float32).at[:R].set(A)
    relcol = jnp.zeros((8, 1), jnp.float32).at[:R, 0].set(relations)
    xui_t, pui_t = _tc_call(gut, git, pt, apad, relcol)
    return (xui_t.reshape(B), pui_t.T)


# speedup vs baseline: 5.4667x; 1.2782x over previous
import jax
import jax.numpy as jnp
from jax import lax
from jax.experimental import pallas as pl
from jax.experimental.pallas import tpu as pltpu

B = 16384
D = 16
R = 5
TILE = 4096


def _tc_body(gut_ref, git_ref, pt_ref, apad_ref, relcol_ref, xui_ref, pui_ref):
    gut = gut_ref[...]
    git = git_ref[...]
    t0 = jnp.dot(pt_ref[0], gut, preferred_element_type=jnp.float32)
    t1 = jnp.dot(pt_ref[1], gut, preferred_element_type=jnp.float32)
    m0 = jnp.sum(t0 * git, axis=0, keepdims=True)
    m1 = jnp.sum(t1 * git, axis=0, keepdims=True)
    mstk = jnp.concatenate([m0, m1], axis=0)
    pstk = jnp.dot(apad_ref[...], mstk, preferred_element_type=jnp.float32)
    valid = lax.broadcasted_iota(jnp.int32, (8, 1), 0) < R
    neg_inf = jnp.float32(float("-inf"))
    mx = jnp.max(jnp.where(valid, pstk, neg_inf), axis=0, keepdims=True)
    ex = jnp.where(valid, jnp.exp(pstk - mx), 0.0)
    den = jnp.sum(ex, axis=0, keepdims=True)
    num = jnp.sum(relcol_ref[...] * ex, axis=0, keepdims=True)
    xui_ref[...] = num / den
    pui_ref[...] = pstk[:R]


@jax.jit
def _tc_call(gut, git, pt, apad, relcol):
    grid = (B // TILE,)
    return pl.pallas_call(
        _tc_body,
        grid=grid,
        in_specs=[
            pl.BlockSpec((D, TILE), lambda b: (0, b)),
            pl.BlockSpec((D, TILE), lambda b: (0, b)),
            pl.BlockSpec((2, D, D), lambda b: (0, 0, 0)),
            pl.BlockSpec((8, 2), lambda b: (0, 0)),
            pl.BlockSpec((8, 1), lambda b: (0, 0)),
        ],
        out_specs=[
            pl.BlockSpec((1, TILE), lambda b: (0, b)),
            pl.BlockSpec((R, TILE), lambda b: (0, b)),
        ],
        out_shape=[
            jax.ShapeDtypeStruct((1, B), jnp.float32),
            jax.ShapeDtypeStruct((R, B), jnp.float32),
        ],
        compiler_params=pltpu.CompilerParams(
            dimension_semantics=("arbitrary",)),
    )(gut, git, pt, apad, relcol)


def kernel(gu, gi, P, A, relations):
    gu = jnp.squeeze(gu)
    gi = jnp.squeeze(gi)
    gut = gu.T
    git = gi.T
    pt = jnp.swapaxes(P, 1, 2)
    apad = jnp.zeros((8, 2), jnp.float32).at[:R].set(A)
    relcol = jnp.zeros((8, 1), jnp.float32).at[:R, 0].set(relations)
    xui_t, pui_t = _tc_call(gut, git, pt, apad, relcol)
    return (xui_t.reshape(B), pui_t.T)


# speedup vs baseline: 6.4861x; 1.1865x over previous
import jax
import jax.numpy as jnp
from jax import lax
from jax.experimental import pallas as pl
from jax.experimental.pallas import tpu as pltpu

B = 16384
D = 16
R = 5
TILE = 8192


def _tc_body(gut_ref, git_ref, pt_ref, apad_ref, relcol_ref, xui_ref, pui_ref):
    gut = gut_ref[...]
    git = git_ref[...]
    t0 = jnp.dot(pt_ref[0], gut, preferred_element_type=jnp.float32)
    t1 = jnp.dot(pt_ref[1], gut, preferred_element_type=jnp.float32)
    m0 = jnp.sum(t0 * git, axis=0, keepdims=True)
    m1 = jnp.sum(t1 * git, axis=0, keepdims=True)
    mstk = jnp.concatenate([m0, m1], axis=0)
    pstk = jnp.dot(apad_ref[...], mstk, preferred_element_type=jnp.float32)
    valid = lax.broadcasted_iota(jnp.int32, (8, 1), 0) < R
    neg_inf = jnp.float32(float("-inf"))
    mx = jnp.max(jnp.where(valid, pstk, neg_inf), axis=0, keepdims=True)
    ex = jnp.where(valid, jnp.exp(pstk - mx), 0.0)
    den = jnp.sum(ex, axis=0, keepdims=True)
    num = jnp.sum(relcol_ref[...] * ex, axis=0, keepdims=True)
    xui_ref[...] = num / den
    pui_ref[...] = pstk[:R]


@jax.jit
def _tc_call(gut, git, pt, apad, relcol):
    grid = (B // TILE,)
    return pl.pallas_call(
        _tc_body,
        grid=grid,
        in_specs=[
            pl.BlockSpec((D, TILE), lambda b: (0, b)),
            pl.BlockSpec((D, TILE), lambda b: (0, b)),
            pl.BlockSpec((2, D, D), lambda b: (0, 0, 0)),
            pl.BlockSpec((8, 2), lambda b: (0, 0)),
            pl.BlockSpec((8, 1), lambda b: (0, 0)),
        ],
        out_specs=[
            pl.BlockSpec((1, TILE), lambda b: (0, b)),
            pl.BlockSpec((R, TILE), lambda b: (0, b)),
        ],
        out_shape=[
            jax.ShapeDtypeStruct((1, B), jnp.float32),
            jax.ShapeDtypeStruct((R, B), jnp.float32),
        ],
        compiler_params=pltpu.CompilerParams(
            dimension_semantics=("arbitrary",)),
    )(gut, git, pt, apad, relcol)


def kernel(gu, gi, P, A, relations):
    gu = jnp.squeeze(gu)
    gi = jnp.squeeze(gi)
    gut = gu.T
    git = gi.T
    pt = jnp.swapaxes(P, 1, 2)
    apad = jnp.zeros((8, 2), jnp.float32).at[:R].set(A)
    relcol = jnp.zeros((8, 1), jnp.float32).at[:R, 0].set(relations)
    xui_t, pui_t = _tc_call(gut, git, pt, apad, relcol)
    return (xui_t.reshape(B), pui_t.T)
